# Initial kernel scaffold; baseline (speedup 1.0000x reference)
import jax
import jax.numpy as jnp
from jax.experimental import pallas as pl


def kernel(z, emb_weight):
    raise NotImplementedError("write your pallas kernel here")


# speedup vs baseline: 2.8490x; 2.8490x over previous
import jax
import jax.numpy as jnp
from jax.experimental import pallas as pl
from jax.experimental.pallas import tpu as pltpu

_KCODES = 512
_EMB = 32
_BETA = 0.25
_BLK = 1024


def _vq_body(z_ref, z2_ref, e_ref, oh_ref, zq_ref, idx_ref, esum_ref,
             dsum_ref, errsum_ref):
    i = pl.program_id(0)
    z = z_ref[...]
    e = e_ref[...]
    z2 = z2_ref[...]
    e2 = jnp.sum(e * e, axis=1)[None, :]
    prod = jax.lax.dot_general(z, e, (((1,), (1,)), ((), ())),
                               preferred_element_type=jnp.float32)
    d = z2 + e2 - 2.0 * prod
    dmin = jnp.min(d, axis=1, keepdims=True)
    lane = jax.lax.broadcasted_iota(jnp.int32, d.shape, 1)
    idx = jnp.min(jnp.where(d == dmin, lane, jnp.int32(_KCODES)), axis=1)
    oh = (jax.lax.broadcasted_iota(jnp.int32, d.shape, 1)
          == idx[:, None]).astype(jnp.float32)
    oh_ref[...] = oh
    zq = jnp.dot(oh, e, preferred_element_type=jnp.float32)
    zq_ref[...] = zq
    idx_ref[0, 0, :] = idx

    @pl.when(i == 0)
    def _():
        esum_ref[...] = jnp.zeros_like(esum_ref)
        dsum_ref[...] = jnp.zeros_like(dsum_ref)
        errsum_ref[...] = jnp.zeros_like(errsum_ref)

    esum_ref[...] += jnp.sum(oh, axis=0)[None, :]
    dsum_ref[...] += jnp.broadcast_to(jnp.sum(d), dsum_ref.shape)
    errsum_ref[...] += jnp.broadcast_to(jnp.sum((zq - z) ** 2),
                                        errsum_ref.shape)


def kernel(z, emb_weight):
    b, c, h, w, dd = z.shape
    n = b * h * w * dd
    nblk = n // _BLK
    z_flat = jnp.transpose(z, (0, 2, 3, 4, 1)).reshape(n, c)
    z2 = (z_flat ** 2).sum(axis=1, keepdims=True)

    oh, zq, idx3, esum, dsum, errsum = pl.pallas_call(
        _vq_body,
        grid=(nblk,),
        in_specs=[
            pl.BlockSpec((_BLK, _EMB), lambda i: (i, 0)),
            pl.BlockSpec((_BLK, 1), lambda i: (i, 0)),
            pl.BlockSpec((_KCODES, _EMB), lambda i: (0, 0)),
        ],
        out_specs=[
            pl.BlockSpec((_BLK, _KCODES), lambda i: (i, 0)),
            pl.BlockSpec((_BLK, _EMB), lambda i: (i, 0)),
            pl.BlockSpec((1, 1, _BLK), lambda i: (i, 0, 0)),
            pl.BlockSpec((1, _KCODES), lambda i: (0, 0)),
            pl.BlockSpec((1, 128), lambda i: (0, 0)),
            pl.BlockSpec((1, 128), lambda i: (0, 0)),
        ],
        out_shape=[
            jax.ShapeDtypeStruct((n, _KCODES), jnp.float32),
            jax.ShapeDtypeStruct((n, _EMB), jnp.float32),
            jax.ShapeDtypeStruct((nblk, 1, _BLK), jnp.int32),
            jax.ShapeDtypeStruct((1, _KCODES), jnp.float32),
            jax.ShapeDtypeStruct((1, 128), jnp.float32),
            jax.ShapeDtypeStruct((1, 128), jnp.float32),
        ],
        compiler_params=pltpu.CompilerParams(
            dimension_semantics=("arbitrary",)),
    )(z_flat, z2, emb_weight)

    min_encoding_indices = idx3.reshape(n, 1)
    mean_distance = dsum[0, 0] / jnp.float32(n * _KCODES)
    loss = (1.0 + _BETA) * errsum[0, 0] / jnp.float32(n * c)
    e_mean = esum[0] / jnp.float32(n)
    perplexity = jnp.exp(-jnp.sum(e_mean * jnp.log(e_mean + 1e-10)))
    z_q_st = jnp.transpose(zq.reshape(b, h, w, dd, c), (0, 4, 1, 2, 3))
    return (z_q_st, loss, perplexity, oh, min_encoding_indices,
            mean_distance)


# speedup vs baseline: 3.1385x; 1.1016x over previous
import jax
import jax.numpy as jnp
from jax.experimental import pallas as pl
from jax.experimental.pallas import tpu as pltpu

_KCODES = 512
_EMB = 32
_BETA = 0.25
_BLK = 1024


def _vq_body(z_ref, z2_ref, en_ref, e_ref, e2_ref, oh_ref, zq_ref, idx_ref,
             esum_ref, zsum_ref, scal_ref):
    i = pl.program_id(0)
    z = z_ref[...]
    z2 = z2_ref[...]
    e2 = e2_ref[...]
    m2 = jax.lax.dot_general(z, en_ref[...], (((1,), (1,)), ((), ())),
                             preferred_element_type=jnp.float32)
    d = (z2 + e2) + m2
    dmin = jnp.min(d, axis=1, keepdims=True)
    lane = jax.lax.broadcasted_iota(jnp.int32, d.shape, 1)
    idx = jnp.min(jnp.where(d == dmin, lane, jnp.int32(_KCODES)), axis=1)
    oh = (lane == idx[:, None]).astype(jnp.float32)
    oh_ref[...] = oh
    zq = jnp.dot(oh, e_ref[...], preferred_element_type=jnp.float32)
    zq_ref[...] = zq
    idx_ref[0, 0, :] = idx

    @pl.when(i == 0)
    def _():
        esum_ref[...] = jnp.zeros_like(esum_ref)
        zsum_ref[...] = jnp.zeros_like(zsum_ref)
        scal_ref[...] = jnp.zeros_like(scal_ref)

    esum_ref[...] += jnp.sum(oh, axis=0)[None, :]
    zsum_ref[...] += jnp.sum(z, axis=0)[None, :]
    scal_ref[...] += (jnp.pad(jnp.sum(dmin).reshape(1, 1), ((0, 0), (0, 127)))
                      + jnp.pad(jnp.sum(z2).reshape(1, 1), ((0, 0), (1, 126))))


def kernel(z, emb_weight):
    b, c, h, w, dd = z.shape
    n = b * h * w * dd
    nblk = n // _BLK
    z_flat = jnp.transpose(z, (0, 2, 3, 4, 1)).reshape(n, c)
    z2 = (z_flat ** 2).sum(axis=1, keepdims=True)
    en = jnp.float32(-2.0) * emb_weight
    e2 = (emb_weight ** 2).sum(axis=1)[None, :]

    oh, zq, idx3, esum, zsum, scal = pl.pallas_call(
        _vq_body,
        grid=(nblk,),
        in_specs=[
            pl.BlockSpec((_BLK, _EMB), lambda i: (i, 0)),
            pl.BlockSpec((_BLK, 1), lambda i: (i, 0)),
            pl.BlockSpec((_KCODES, _EMB), lambda i: (0, 0)),
            pl.BlockSpec((_KCODES, _EMB), lambda i: (0, 0)),
            pl.BlockSpec((1, _KCODES), lambda i: (0, 0)),
        ],
        out_specs=[
            pl.BlockSpec((_BLK, _KCODES), lambda i: (i, 0)),
            pl.BlockSpec((_BLK, _EMB), lambda i: (i, 0)),
            pl.BlockSpec((1, 1, _BLK), lambda i: (i, 0, 0)),
            pl.BlockSpec((1, _KCODES), lambda i: (0, 0)),
            pl.BlockSpec((1, _EMB), lambda i: (0, 0)),
            pl.BlockSpec((1, 128), lambda i: (0, 0)),
        ],
        out_shape=[
            jax.ShapeDtypeStruct((n, _KCODES), jnp.float32),
            jax.ShapeDtypeStruct((n, _EMB), jnp.float32),
            jax.ShapeDtypeStruct((nblk, 1, _BLK), jnp.int32),
            jax.ShapeDtypeStruct((1, _KCODES), jnp.float32),
            jax.ShapeDtypeStruct((1, _EMB), jnp.float32),
            jax.ShapeDtypeStruct((1, 128), jnp.float32),
        ],
        compiler_params=pltpu.CompilerParams(
            dimension_semantics=("arbitrary",)),
    )(z_flat, z2, en, emb_weight, e2)

    min_encoding_indices = idx3.reshape(n, 1)
    dminsum = scal[0, 0]
    z2sum = scal[0, 1]
    dsum = (jnp.float32(_KCODES) * z2sum + jnp.float32(n) * jnp.sum(e2)
            - 2.0 * jnp.sum(zsum[0] * jnp.sum(emb_weight, axis=0)))
    mean_distance = dsum / jnp.float32(n * _KCODES)
    loss = (1.0 + _BETA) * dminsum / jnp.float32(n * c)
    e_mean = esum[0] / jnp.float32(n)
    perplexity = jnp.exp(-jnp.sum(e_mean * jnp.log(e_mean + 1e-10)))
    z_q_st = jnp.transpose(zq.reshape(b, h, w, dd, c), (0, 4, 1, 2, 3))
    return (z_q_st, loss, perplexity, oh, min_encoding_indices,
            mean_distance)


# speedup vs baseline: 3.6399x; 1.1598x over previous
import jax
import jax.numpy as jnp
from jax.experimental import pallas as pl
from jax.experimental.pallas import tpu as pltpu

_KCODES = 512
_EMB = 32
_BETA = 0.25
_HB = 8
_WB = 256


def _vq_body(zc_ref, z2_ref, en_ref, e_ref, e2_ref, oh_ref, zq_ref, idx_ref,
             esum_ref, zsum_ref, scal_ref):
    first = ((pl.program_id(0) == 0) & (pl.program_id(1) == 0)
             & (pl.program_id(2) == 0))

    @pl.when(first)
    def _():
        esum_ref[...] = jnp.zeros_like(esum_ref)
        zsum_ref[...] = jnp.zeros_like(zsum_ref)
        scal_ref[...] = jnp.zeros_like(scal_ref)

    en = en_ref[...]
    e = e_ref[...]
    e2 = e2_ref[...]
    for g in range(_HB):
        zg = zc_ref[0, :, g, :]
        z2g = z2_ref[0, g, :]
        m2 = jax.lax.dot_general(zg, en, (((0,), (1,)), ((), ())),
                                 preferred_element_type=jnp.float32)
        d = (z2g[:, None] + e2) + m2
        dmin = jnp.min(d, axis=1, keepdims=True)
        lane = jax.lax.broadcasted_iota(jnp.int32, d.shape, 1)
        idx = jnp.min(jnp.where(d == dmin, lane, jnp.int32(_KCODES)), axis=1)
        oh = (lane == idx[:, None]).astype(jnp.float32)
        oh_ref[0, g, :, :] = oh
        zqt = jax.lax.dot_general(e, oh, (((0,), (1,)), ((), ())),
                                  preferred_element_type=jnp.float32)
        zq_ref[0, :, g, :] = zqt
        idx_ref[0, g, :] = idx
        esum_ref[...] += jnp.sum(oh, axis=0)[None, :]
        zsum_ref[...] += zg
        scal_ref[...] += (
            jnp.pad(jnp.sum(dmin).reshape(1, 1), ((0, 0), (0, 127)))
            + jnp.pad(jnp.sum(z2g).reshape(1, 1), ((0, 0), (1, 126))))


def kernel(z, emb_weight):
    b, c, h, w, dd = z.shape
    n = b * h * w * dd
    wd = w * dd
    zc = z.reshape(b, c, h, wd)
    z2 = (z ** 2).sum(axis=1).reshape(b, h, wd)
    en = jnp.float32(-2.0) * emb_weight
    e2 = (emb_weight ** 2).sum(axis=1)[None, :]

    grid = (b, h // _HB, wd // _WB)
    oh, zqt, idx3, esum, zsum, scal = pl.pallas_call(
        _vq_body,
        grid=grid,
        in_specs=[
            pl.BlockSpec((1, c, _HB, _WB), lambda i, j, k: (i, 0, j, k)),
            pl.BlockSpec((1, _HB, _WB), lambda i, j, k: (i, j, k)),
            pl.BlockSpec((_KCODES, _EMB), lambda i, j, k: (0, 0)),
            pl.BlockSpec((_KCODES, _EMB), lambda i, j, k: (0, 0)),
            pl.BlockSpec((1, _KCODES), lambda i, j, k: (0, 0)),
        ],
        out_specs=[
            pl.BlockSpec((1, _HB, _WB, _KCODES), lambda i, j, k: (i, j, k, 0)),
            pl.BlockSpec((1, c, _HB, _WB), lambda i, j, k: (i, 0, j, k)),
            pl.BlockSpec((1, _HB, _WB), lambda i, j, k: (i, j, k)),
            pl.BlockSpec((1, _KCODES), lambda i, j, k: (0, 0)),
            pl.BlockSpec((_EMB, _WB), lambda i, j, k: (0, 0)),
            pl.BlockSpec((1, 128), lambda i, j, k: (0, 0)),
        ],
        out_shape=[
            jax.ShapeDtypeStruct((b, h, wd, _KCODES), jnp.float32),
            jax.ShapeDtypeStruct((b, c, h, wd), jnp.float32),
            jax.ShapeDtypeStruct((b, h, wd), jnp.int32),
            jax.ShapeDtypeStruct((1, _KCODES), jnp.float32),
            jax.ShapeDtypeStruct((_EMB, _WB), jnp.float32),
            jax.ShapeDtypeStruct((1, 128), jnp.float32),
        ],
        compiler_params=pltpu.CompilerParams(
            dimension_semantics=("arbitrary", "arbitrary", "arbitrary")),
    )(zc, z2, en, emb_weight, e2)

    min_encodings = oh.reshape(n, _KCODES)
    min_encoding_indices = idx3.reshape(n, 1)
    z_q_st = zqt.reshape(b, c, h, w, dd)
    dminsum = scal[0, 0]
    z2sum = scal[0, 1]
    dsum = (jnp.float32(_KCODES) * z2sum + jnp.float32(n) * jnp.sum(e2)
            - 2.0 * jnp.sum(jnp.sum(zsum, axis=1)
                            * jnp.sum(emb_weight, axis=0)))
    mean_distance = dsum / jnp.float32(n * _KCODES)
    loss = (1.0 + _BETA) * dminsum / jnp.float32(n * c)
    e_mean = esum[0] / jnp.float32(n)
    perplexity = jnp.exp(-jnp.sum(e_mean * jnp.log(e_mean + 1e-10)))
    return (z_q_st, loss, perplexity, min_encodings, min_encoding_indices,
            mean_distance)
